# baseline (speedup 1.0000x reference)
import jax
import jax.numpy as jnp
from jax.experimental import pallas as pl
from jax.experimental.pallas import tpu as pltpu

_CH = 1024
_NBUF = 5


def _make(B, L, D):
    n_chunks = B * L // _CH
    n_tchunks = L // _CH

    def body(x_hbm, t_hbm, o_hbm, tbuf, xbuf, obuf, sem_t, sem_x, sem_o):
        def t_copy(j):
            return pltpu.make_async_copy(
                t_hbm.at[pl.ds(j * _CH, _CH)],
                tbuf.at[pl.ds(j * _CH, _CH)],
                sem_t.at[j],
            )

        def x_copy(i):
            return pltpu.make_async_copy(
                x_hbm.at[pl.ds(i * _CH, _CH)], xbuf.at[i % _NBUF], sem_x.at[i % _NBUF]
            )

        def o_copy(i):
            return pltpu.make_async_copy(
                obuf.at[i % _NBUF], o_hbm.at[pl.ds(i * _CH, _CH)], sem_o.at[i % _NBUF]
            )

        t_copy(0).start()
        x_copy(0).start()
        for j in range(1, n_tchunks):
            t_copy(j).start()
        for i in range(1, min(_NBUF, n_chunks)):
            x_copy(i).start()

        for i in range(n_chunks):
            j = i % n_tchunks
            x_copy(i).wait()
            if i < n_tchunks:
                t_copy(j).wait()
            if i >= _NBUF:
                o_copy(i - _NBUF).wait()
            obuf[i % _NBUF, :, :] = xbuf[i % _NBUF, :, :] + tbuf[pl.ds(j * _CH, _CH), :]
            o_copy(i).start()
            if i + _NBUF < n_chunks:
                x_copy(i + _NBUF).start()
        for i in range(max(0, n_chunks - _NBUF), n_chunks):
            o_copy(i).wait()

    return body


def kernel(x, row_embed):
    B, L, D = x.shape
    table = row_embed[:L]
    x2 = x.reshape(B * L, D)
    out = pl.pallas_call(
        _make(B, L, D),
        in_specs=[
            pl.BlockSpec(memory_space=pltpu.MemorySpace.HBM),
            pl.BlockSpec(memory_space=pltpu.MemorySpace.HBM),
        ],
        out_specs=pl.BlockSpec(memory_space=pltpu.MemorySpace.HBM),
        out_shape=jax.ShapeDtypeStruct((B * L, D), x.dtype),
        scratch_shapes=[
            pltpu.VMEM((L, D), jnp.float32),
            pltpu.VMEM((_NBUF, _CH, D), jnp.float32),
            pltpu.VMEM((_NBUF, _CH, D), jnp.float32),
            pltpu.SemaphoreType.DMA((L // _CH,)),
            pltpu.SemaphoreType.DMA((_NBUF,)),
            pltpu.SemaphoreType.DMA((_NBUF,)),
        ],
    )(x2, table)
    return out.reshape(B, L, D)


# speedup vs baseline: 1.0079x; 1.0079x over previous
import jax
import jax.numpy as jnp
from jax.experimental import pallas as pl
from jax.experimental.pallas import tpu as pltpu

_CH = 1024
_NBUF = 6


def _make(B, L, D):
    n_chunks = B * L // _CH
    n_tchunks = L // _CH

    def body(x_hbm, t_hbm, o_hbm, tbuf, xbuf, obuf, sem_t, sem_x, sem_o):
        def t_copy(j):
            return pltpu.make_async_copy(
                t_hbm.at[pl.ds(j * _CH, _CH)],
                tbuf.at[pl.ds(j * _CH, _CH)],
                sem_t.at[j],
            )

        def x_copy(i):
            return pltpu.make_async_copy(
                x_hbm.at[pl.ds(i * _CH, _CH)], xbuf.at[i % _NBUF], sem_x.at[i % _NBUF]
            )

        def o_copy(i):
            return pltpu.make_async_copy(
                obuf.at[i % _NBUF], o_hbm.at[pl.ds(i * _CH, _CH)], sem_o.at[i % _NBUF]
            )

        t_copy(0).start()
        x_copy(0).start()
        for j in range(1, n_tchunks):
            t_copy(j).start()
        for i in range(1, min(_NBUF, n_chunks)):
            x_copy(i).start()

        for i in range(n_chunks):
            j = i % n_tchunks
            x_copy(i).wait()
            if i < n_tchunks:
                t_copy(j).wait()
            if i >= _NBUF:
                o_copy(i - _NBUF).wait()
            obuf[i % _NBUF, :, :] = xbuf[i % _NBUF, :, :] + tbuf[pl.ds(j * _CH, _CH), :]
            o_copy(i).start()
            if i + _NBUF < n_chunks:
                x_copy(i + _NBUF).start()
        for i in range(max(0, n_chunks - _NBUF), n_chunks):
            o_copy(i).wait()

    return body


def kernel(x, row_embed):
    B, L, D = x.shape
    table = row_embed[:L]
    x2 = x.reshape(B * L, D)
    out = pl.pallas_call(
        _make(B, L, D),
        in_specs=[
            pl.BlockSpec(memory_space=pltpu.MemorySpace.HBM),
            pl.BlockSpec(memory_space=pltpu.MemorySpace.HBM),
        ],
        out_specs=pl.BlockSpec(memory_space=pltpu.MemorySpace.HBM),
        out_shape=jax.ShapeDtypeStruct((B * L, D), x.dtype),
        scratch_shapes=[
            pltpu.VMEM((L, D), jnp.float32),
            pltpu.VMEM((_NBUF, _CH, D), jnp.float32),
            pltpu.VMEM((_NBUF, _CH, D), jnp.float32),
            pltpu.SemaphoreType.DMA((L // _CH,)),
            pltpu.SemaphoreType.DMA((_NBUF,)),
            pltpu.SemaphoreType.DMA((_NBUF,)),
        ],
    )(x2, table)
    return out.reshape(B, L, D)
